# baseline (speedup 1.0000x reference)
import functools

import jax
import jax.numpy as jnp
from jax import lax
from jax.experimental import pallas as pl
from jax.experimental.pallas import tpu as pltpu
from jax.experimental.pallas import tpu_sc as plsc

DIM = 32
BATCH = 16384
PACK = 128 // DIM

_INFO = plsc.get_sparse_core_info()
_NC, _NS, _L = _INFO.num_cores, _INFO.num_subcores, _INFO.num_lanes
_NW = _NC * _NS
_BPW = BATCH // _NW
_CHUNK = 128
_NCHUNK = _BPW // _CHUNK

_mesh = plsc.VectorSubcoreMesh(core_axis_name="c", subcore_axis_name="s")


@functools.partial(
    pl.kernel,
    mesh=_mesh,
    out_type=jax.ShapeDtypeStruct((BATCH,), jnp.float32),
    compiler_params=pltpu.CompilerParams(needs_layout_passes=False),
    scratch_types=[
        pltpu.VMEM((_NCHUNK, _CHUNK), jnp.int32),
        pltpu.VMEM((_NCHUNK, _CHUNK), jnp.int32),
        pltpu.VMEM((_NCHUNK, _CHUNK), jnp.int32),
        pltpu.VMEM((_NCHUNK, _CHUNK), jnp.int32),
        pltpu.VMEM((_NCHUNK, _CHUNK), jnp.int32),
        pltpu.VMEM((_NCHUNK, _CHUNK), jnp.int32),
        pltpu.VMEM((2, _CHUNK, 128), jnp.float32),
        pltpu.VMEM((2, _CHUNK, 128), jnp.float32),
        pltpu.VMEM((2, _CHUNK, 128), jnp.float32),
        pltpu.VMEM((_BPW,), jnp.float32),
        pltpu.SemaphoreType.DMA,
        pltpu.SemaphoreType.DMA,
    ],
)
def _bpr_sc(ui_hbm, pi_hbm, ni_hbm, gu_hbm, gi_hbm, out_hbm,
            ui_v, pi_v, ni_v, upk_v, ppk_v, npk_v,
            u_rows, p_rows, n_rows, out_v, sem0, sem1):
    wid = lax.axis_index("s") * _NC + lax.axis_index("c")
    base = wid * _BPW
    sems = (sem0, sem1)

    pltpu.sync_copy(ui_hbm.at[wid], ui_v)
    pltpu.sync_copy(pi_hbm.at[wid], pi_v)
    pltpu.sync_copy(ni_hbm.at[wid], ni_v)

    for c in range(_NCHUNK):
        for g in range(_CHUNK // _L):
            s = pl.ds(g * _L, _L)
            upk_v[c, s] = lax.shift_right_logical(ui_v[c, s], 2)
            ppk_v[c, s] = lax.shift_right_logical(pi_v[c, s], 2)
            npk_v[c, s] = lax.shift_right_logical(ni_v[c, s], 2)

    def fire(c):
        b = c % 2
        sem = sems[b]
        return (pltpu.async_copy(gu_hbm.at[upk_v.at[c]], u_rows.at[b], sem),
                pltpu.async_copy(gi_hbm.at[ppk_v.at[c]], p_rows.at[b], sem),
                pltpu.async_copy(gi_hbm.at[npk_v.at[c]], n_rows.at[b], sem))

    lanes = lax.iota(jnp.int32, _L)
    inflight = fire(0)

    for c in range(_NCHUNK):
        b = c % 2
        cur = inflight
        if c + 1 < _NCHUNK:
            inflight = fire(c + 1)
        for cp in cur:
            cp.wait()

        ub, pb, nb = u_rows.at[b], p_rows.at[b], n_rows.at[b]

        def group(g, _, c=c, ub=ub, pb=pb, nb=nb):
            s = pl.ds(g * _L, _L)
            colu = (ui_v[c, s] & 3) * DIM
            colp = (pi_v[c, s] & 3) * DIM
            coln = (ni_v[c, s] & 3) * DIM
            rowv = g * _L + lanes
            acc = jnp.zeros((_L,), jnp.float32)
            for d in range(DIM):
                u = plsc.load_gather(ub, [rowv, colu + d])
                p = plsc.load_gather(pb, [rowv, colp + d])
                n = plsc.load_gather(nb, [rowv, coln + d])
                acc = acc + u * (p - n)
            out_v[pl.ds(c * _CHUNK + g * _L, _L)] = acc
            return 0

        lax.fori_loop(0, _CHUNK // _L, group, 0)

    pltpu.sync_copy(out_v, out_hbm.at[pl.ds(base, _BPW)])


def kernel(ui, pi, ni, gamma_users, gamma_items):
    n_users, dim = gamma_users.shape
    n_items, _ = gamma_items.shape
    gu2 = gamma_users.reshape(n_users * dim // 128, 128)
    gi2 = gamma_items.reshape(n_items * dim // 128, 128)
    ui3 = ui.astype(jnp.int32).reshape(_NW, _NCHUNK, _CHUNK)
    pi3 = pi.astype(jnp.int32).reshape(_NW, _NCHUNK, _CHUNK)
    ni3 = ni.astype(jnp.int32).reshape(_NW, _NCHUNK, _CHUNK)
    return _bpr_sc(ui3, pi3, ni3, gu2, gi2)
